# Initial kernel scaffold; baseline (speedup 1.0000x reference)
import jax
import jax.numpy as jnp
from jax.experimental import pallas as pl


def kernel(x, codebooks):
    raise NotImplementedError("write your pallas kernel here")


# speedup vs baseline: 1.5241x; 1.5241x over previous
import functools

import jax
import jax.numpy as jnp
from jax import lax
from jax.experimental import pallas as pl
from jax.experimental.pallas import tpu as pltpu
from jax.experimental.pallas import tpu_sc as plsc

EPS = 1e-12
ROWS = 256
PREC = lax.Precision.DEFAULT

NC = 2
NS = 16
NW = NC * NS
GCH = 128


def _norm_codebook(cb):
    norms = jnp.sqrt(jnp.sum(cb * cb, axis=1, keepdims=True))
    return cb / (norms + EPS)


def _sim_argmax(rn, cn, k):
    sim = lax.dot_general(rn, cn, (((1,), (1,)), ((), ())),
                          preferred_element_type=jnp.float32,
                          precision=PREC)
    m = jnp.max(sim, axis=1)
    iota = lax.broadcasted_iota(jnp.int32, sim.shape, 1)
    idx = jnp.min(jnp.where(sim == m[:, None], iota, k), axis=1)
    return m, idx


def _first_body(x_ref, cb_ref, idx_ref, scal_ref, cn_ref):
    i = pl.program_id(0)
    k = cb_ref.shape[0]

    @pl.when(i == 0)
    def _():
        cn_ref[...] = _norm_codebook(cb_ref[...])

    r = x_ref[...]
    sumsq = jnp.sum(r * r, axis=1, keepdims=True)
    rn = r / (jnp.sqrt(sumsq) + EPS)
    m, idx = _sim_argmax(rn, cn_ref[...], k)
    idx_ref[0, 0, :] = idx
    scal_ref[0, 0, :] = m


def _level_body(prev_ref, q_ref, cb_ref, res_ref, idx_ref, scal_ref, s_ref,
                cn_ref):
    i = pl.program_id(0)
    k = cb_ref.shape[0]

    @pl.when(i == 0)
    def _():
        cn_ref[...] = _norm_codebook(cb_ref[...])
        s_ref[0, 0] = 0.0

    r = prev_ref[...] - q_ref[...]
    res_ref[...] = r
    sumsq = jnp.sum(r * r, axis=1, keepdims=True)
    rn = r / (jnp.sqrt(sumsq) + EPS)
    m, idx = _sim_argmax(rn, cn_ref[...], k)
    idx_ref[0, 0, :] = idx
    scal_ref[0, 0, :] = m
    s_ref[0, 0] += jnp.sum(sumsq)


def _fin_body(flat_ref, res3_ref, q3_ref, s1_ref, s2_ref, s3_ref,
              xq_ref, loss_ref, *, inv_count):
    i = pl.program_id(0)

    @pl.when(i == 0)
    def _():
        loss_ref[0, 0] = 0.0

    r4 = res3_ref[...] - q3_ref[...]
    xq_ref[...] = flat_ref[...] - r4
    loss_ref[0, 0] += jnp.sum(r4 * r4)

    @pl.when(i == pl.num_programs(0) - 1)
    def _():
        total = s1_ref[0, 0] + s2_ref[0, 0] + s3_ref[0, 0] + loss_ref[0, 0]
        loss_ref[0, 0] = total * inv_count


def _row_spec(rows, d):
    return pl.BlockSpec((rows, d), lambda i: (i, 0))


def _lane_spec(rows):
    return pl.BlockSpec((1, 1, rows), lambda i: (i, 0, 0))


def _scalar_spec():
    return pl.BlockSpec((1, 1), lambda i: (0, 0), memory_space=pltpu.SMEM)


def _level_first(flat, cb):
    n, d = flat.shape
    k = cb.shape[0]
    nt = n // ROWS
    idx3, scal3 = pl.pallas_call(
        _first_body,
        grid=(nt,),
        in_specs=[_row_spec(ROWS, d), pl.BlockSpec((k, d), lambda i: (0, 0))],
        out_specs=[_lane_spec(ROWS), _lane_spec(ROWS)],
        out_shape=[
            jax.ShapeDtypeStruct((nt, 1, ROWS), jnp.int32),
            jax.ShapeDtypeStruct((nt, 1, ROWS), jnp.float32),
        ],
        scratch_shapes=[pltpu.VMEM((k, d), jnp.float32)],
        compiler_params=pltpu.CompilerParams(
            dimension_semantics=("arbitrary",)),
    )(flat, cb)
    return idx3.reshape(n), scal3.reshape(n)


def _level_next(prev, q, cb):
    n, d = prev.shape
    k = cb.shape[0]
    nt = n // ROWS
    res, idx3, scal3, s = pl.pallas_call(
        _level_body,
        grid=(nt,),
        in_specs=[_row_spec(ROWS, d), _row_spec(ROWS, d),
                  pl.BlockSpec((k, d), lambda i: (0, 0))],
        out_specs=[_row_spec(ROWS, d), _lane_spec(ROWS), _lane_spec(ROWS),
                   _scalar_spec()],
        out_shape=[
            jax.ShapeDtypeStruct((n, d), jnp.float32),
            jax.ShapeDtypeStruct((nt, 1, ROWS), jnp.int32),
            jax.ShapeDtypeStruct((nt, 1, ROWS), jnp.float32),
            jax.ShapeDtypeStruct((1, 1), jnp.float32),
        ],
        scratch_shapes=[pltpu.VMEM((k, d), jnp.float32)],
        compiler_params=pltpu.CompilerParams(
            dimension_semantics=("arbitrary",)),
    )(prev, q, cb)
    return res, idx3.reshape(n), scal3.reshape(n), s


def _finalize(flat, res3, q3, s1, s2, s3):
    n, d = flat.shape
    nt = n // ROWS
    inv_count = 1.25 / (4.0 * n * d)
    xq, loss = pl.pallas_call(
        functools.partial(_fin_body, inv_count=inv_count),
        grid=(nt,),
        in_specs=[_row_spec(ROWS, d), _row_spec(ROWS, d), _row_spec(ROWS, d),
                  _scalar_spec(), _scalar_spec(), _scalar_spec()],
        out_specs=[_row_spec(ROWS, d), _scalar_spec()],
        out_shape=[
            jax.ShapeDtypeStruct((n, d), jnp.float32),
            jax.ShapeDtypeStruct((1, 1), jnp.float32),
        ],
        compiler_params=pltpu.CompilerParams(
            dimension_semantics=("arbitrary",)),
    )(flat, res3, q3, s1, s2, s3)
    return xq, loss.reshape(())


def _sc_gather(table, idx, n, d):
    per_w = n // NW
    ch = per_w // GCH
    idx3 = idx.reshape(NW, ch, GCH)
    mesh = plsc.VectorSubcoreMesh(core_axis_name="c", subcore_axis_name="s")

    @functools.partial(
        pl.kernel,
        out_type=jax.ShapeDtypeStruct((n, d), jnp.float32),
        mesh=mesh,
        scratch_types=[
            pltpu.VMEM((GCH,), jnp.int32),
            pltpu.VMEM((GCH, d), jnp.float32),
            pltpu.SemaphoreType.DMA,
        ],
    )
    def gather(table_hbm, idx_hbm, out_hbm, idx_v, rows_v, sem):
        wid = lax.axis_index("s") * NC + lax.axis_index("c")
        base = wid * per_w
        for c in range(ch):
            pltpu.sync_copy(idx_hbm.at[wid, c], idx_v)
            pltpu.async_copy(table_hbm.at[idx_v], rows_v, sem).wait()
            pltpu.sync_copy(rows_v, out_hbm.at[pl.ds(base + c * GCH, GCH)])

    return gather(table, idx3)


def kernel(x, codebooks):
    b, t, d = x.shape
    n_levels, k, _ = codebooks.shape
    n = b * t
    flat = x.reshape(n, d)

    idx0, scal0 = _level_first(flat, codebooks[0])
    q0 = _sc_gather(codebooks[0], idx0, n, d)
    res1, idx1, scal1, s1 = _level_next(flat, q0, codebooks[1])
    q1 = _sc_gather(codebooks[1], idx1, n, d)
    res2, idx2, scal2, s2 = _level_next(res1, q1, codebooks[2])
    q2 = _sc_gather(codebooks[2], idx2, n, d)
    res3, idx3, scal3, s3 = _level_next(res2, q2, codebooks[3])
    q3 = _sc_gather(codebooks[3], idx3, n, d)
    x_q, mean_loss = _finalize(flat, res3, q3, s1, s2, s3)

    x_q = x_q.reshape(b, t, d)
    all_indices = jnp.stack([idx0, idx1, idx2, idx3], axis=-1)
    all_indices = all_indices.reshape(b, t, n_levels)
    all_scalars = jnp.stack([scal0, scal1, scal2, scal3], axis=-1)
    all_scalars = all_scalars.reshape(b, t, n_levels)
    return (x_q, mean_loss, all_indices, all_scalars)
